# baseline (speedup 1.0000x reference)
import jax
import jax.numpy as jnp
from jax.experimental import pallas as pl
from jax.experimental.pallas import tpu as pltpu

B = 16
T = 2048
D = 256
H = 256
TC = 128


def _lstm_kernel(len_ref, x_ref, w_in_ref, b_in_ref, w_st_ref,
                 y_ref, hN_ref, cN_ref, pi_s, h_s, c_s):
    i = pl.program_id(0)

    @pl.when(i == 0)
    def _init():
        h_s[:] = jnp.zeros((B, H), jnp.float32)
        c_s[:] = jnp.zeros((B, H), jnp.float32)

    x = jnp.swapaxes(x_ref[:], 0, 1).reshape(TC * B, D)
    pi_s[:] = (jnp.dot(x, w_in_ref[:], preferred_element_type=jnp.float32)
               + b_in_ref[:])

    lens = len_ref[:]
    w_st = w_st_ref[:]

    def sig(v):
        return 0.5 * jnp.tanh(0.5 * v) + 0.5

    def step(t, carry):
        h, c = carry
        pi_t = pi_s[pl.ds(t * B, B), :]
        hb = h.astype(jnp.bfloat16)
        ps = (jnp.dot(hb[:, :128], w_st[:128, :],
                      preferred_element_type=jnp.float32)
              + jnp.dot(hb[:, 128:], w_st[128:, :],
                        preferred_element_type=jnp.float32))
        g = pi_t[:, : 5 * H] + ps
        input_gate = sig(g[:, 0 * H:1 * H])
        forget_gate = sig(g[:, 1 * H:2 * H])
        memory_init = jnp.tanh(g[:, 2 * H:3 * H])
        output_gate = sig(g[:, 3 * H:4 * H])
        highway_gate = sig(g[:, 4 * H:5 * H])
        memory = input_gate * memory_init + forget_gate * c
        out = output_gate * jnp.tanh(memory)
        out = highway_gate * out + (1.0 - highway_gate) * pi_t[:, 5 * H:6 * H]
        m2 = (i * TC + t) < lens
        c_new = jnp.where(m2, memory, c)
        h_new = jnp.where(m2, out, h)
        y_ref[:, pl.ds(t, 1), :] = jnp.where(m2, out, 0.0).reshape(B, 1, H)
        return h_new, c_new

    h_fin, c_fin = jax.lax.fori_loop(0, TC, step, (h_s[:], c_s[:]))
    h_s[:] = h_fin
    c_s[:] = c_fin
    hN_ref[:] = h_fin
    cN_ref[:] = c_fin


@jax.jit
def kernel(sequence, batch_lengths, W_in, b_in, W_state, b_state):
    lens = batch_lengths.astype(jnp.int32).reshape(B, 1)
    w_in_t = W_in.T
    w_st_t = W_state.T.astype(jnp.bfloat16)
    b_all = (b_in.at[:5 * H].add(b_state)).reshape(1, 6 * H)

    grid = (T // TC,)
    y, hN, cN = pl.pallas_call(
        _lstm_kernel,
        grid=grid,
        in_specs=[
            pl.BlockSpec((B, 1), lambda i: (0, 0)),
            pl.BlockSpec((B, TC, D), lambda i: (0, i, 0)),
            pl.BlockSpec((D, 6 * H), lambda i: (0, 0)),
            pl.BlockSpec((1, 6 * H), lambda i: (0, 0)),
            pl.BlockSpec((H, 5 * H), lambda i: (0, 0)),
        ],
        out_specs=[
            pl.BlockSpec((B, TC, H), lambda i: (0, i, 0)),
            pl.BlockSpec((B, H), lambda i: (0, 0)),
            pl.BlockSpec((B, H), lambda i: (0, 0)),
        ],
        out_shape=[
            jax.ShapeDtypeStruct((B, T, H), jnp.float32),
            jax.ShapeDtypeStruct((B, H), jnp.float32),
            jax.ShapeDtypeStruct((B, H), jnp.float32),
        ],
        scratch_shapes=[
            pltpu.VMEM((TC * B, 6 * H), jnp.float32),
            pltpu.VMEM((B, H), jnp.float32),
            pltpu.VMEM((B, H), jnp.float32),
        ],
    )(lens, sequence, w_in_t, b_all, w_st_t)

    return y, hN[None], cN[None]


# speedup vs baseline: 1.0052x; 1.0052x over previous
import jax
import jax.numpy as jnp
from jax.experimental import pallas as pl
from jax.experimental.pallas import tpu as pltpu

B = 16
T = 2048
D = 256
H = 256
TC = 128


def _lstm_kernel(len_ref, x_ref, w_in_ref, b_in_ref, w_st_ref,
                 y_ref, hN_ref, cN_ref, pi_s, h_s, c_s):
    i = pl.program_id(0)

    @pl.when(i == 0)
    def _init():
        h_s[:] = jnp.zeros((B, H), jnp.float32)
        c_s[:] = jnp.zeros((B, H), jnp.float32)

    x = jnp.swapaxes(x_ref[:], 0, 1).reshape(TC * B, D)
    pi_s[:] = (jnp.dot(x, w_in_ref[:], preferred_element_type=jnp.float32)
               + b_in_ref[:])

    lens = len_ref[:]
    w_st = w_st_ref[:]

    def sig(v):
        return 0.5 * jnp.tanh(0.5 * v) + 0.5

    def step(t, carry):
        h, c = carry
        pi_t = pi_s[pl.ds(t * B, B), :]
        hb = h.astype(jnp.bfloat16)
        ps = (jnp.dot(hb[:, :128], w_st[:128, :],
                      preferred_element_type=jnp.float32)
              + jnp.dot(hb[:, 128:], w_st[128:, :],
                        preferred_element_type=jnp.float32))
        g = pi_t[:, : 5 * H] + ps
        input_gate = sig(g[:, 0 * H:1 * H])
        forget_gate = sig(g[:, 1 * H:2 * H])
        memory_init = jnp.tanh(g[:, 2 * H:3 * H])
        output_gate = sig(g[:, 3 * H:4 * H])
        highway_gate = sig(g[:, 4 * H:5 * H])
        memory = input_gate * memory_init + forget_gate * c
        out = ((output_gate * highway_gate) * jnp.tanh(memory)
               + (1.0 - highway_gate) * pi_t[:, 5 * H:6 * H])
        m2 = (i * TC + t) < lens
        c_new = jnp.where(m2, memory, c)
        h_new = jnp.where(m2, out, h)
        y_ref[:, pl.ds(t, 1), :] = jnp.where(m2, out, 0.0).reshape(B, 1, H)
        return h_new, c_new

    h_fin, c_fin = jax.lax.fori_loop(0, TC, step, (h_s[:], c_s[:]))
    h_s[:] = h_fin
    c_s[:] = c_fin
    hN_ref[:] = h_fin
    cN_ref[:] = c_fin


@jax.jit
def kernel(sequence, batch_lengths, W_in, b_in, W_state, b_state):
    lens = batch_lengths.astype(jnp.int32).reshape(B, 1)
    w_in_t = W_in.T
    w_st_t = W_state.T.astype(jnp.bfloat16)
    b_all = (b_in.at[:5 * H].add(b_state)).reshape(1, 6 * H)

    grid = (T // TC,)
    y, hN, cN = pl.pallas_call(
        _lstm_kernel,
        grid=grid,
        in_specs=[
            pl.BlockSpec((B, 1), lambda i: (0, 0)),
            pl.BlockSpec((B, TC, D), lambda i: (0, i, 0)),
            pl.BlockSpec((D, 6 * H), lambda i: (0, 0)),
            pl.BlockSpec((1, 6 * H), lambda i: (0, 0)),
            pl.BlockSpec((H, 5 * H), lambda i: (0, 0)),
        ],
        out_specs=[
            pl.BlockSpec((B, TC, H), lambda i: (0, i, 0)),
            pl.BlockSpec((B, H), lambda i: (0, 0)),
            pl.BlockSpec((B, H), lambda i: (0, 0)),
        ],
        out_shape=[
            jax.ShapeDtypeStruct((B, T, H), jnp.float32),
            jax.ShapeDtypeStruct((B, H), jnp.float32),
            jax.ShapeDtypeStruct((B, H), jnp.float32),
        ],
        scratch_shapes=[
            pltpu.VMEM((TC * B, 6 * H), jnp.float32),
            pltpu.VMEM((B, H), jnp.float32),
            pltpu.VMEM((B, H), jnp.float32),
        ],
    )(lens, sequence, w_in_t, b_all, w_st_t)

    return y, hN[None], cN[None]
